# baseline (speedup 1.0000x reference)
import functools

import jax
import jax.numpy as jnp
from jax import lax
from jax.experimental import pallas as pl
from jax.experimental.pallas import tpu as pltpu
from jax.experimental.pallas import tpu_sc as plsc

B = 4096
NEG = 200
D = 32
USER_NUM = 100000
ITEM_NUM = 1000000
WEIGHT = 0.5
MARGIN1 = 4.0
MARGIN2 = 0.5
GAMMA = 1e-4

NC = 2
NS = 16
NW = NC * NS
UPW = B // NW
CH_USERS = 4
CH_ROWS = CH_USERS * NEG
NCH = UPW // CH_USERS

IT_ROWS_PW = 31248
IT_SWEEP_ROWS = 112
IT_CHUNKS = IT_ROWS_PW // IT_SWEEP_ROWS
IT_TAIL_BASE = IT_ROWS_PW * NW
IT_TAIL = ITEM_NUM - IT_TAIL_BASE
UT_ROWS_PW = 3120
UT_SWEEP_ROWS = 104
UT_CHUNKS = UT_ROWS_PW // UT_SWEEP_ROWS
UT_TAIL_BASE = UT_ROWS_PW * NW
UT_TAIL = USER_NUM - UT_TAIL_BASE
SWEEP_BUF_ROWS = 160

ID_PW = 31248
ID_CHUNK = 1008
ID_CHUNKS = ID_PW // ID_CHUNK
ID_TAIL_BASE = ID_PW * NW
ID_TAIL = ITEM_NUM - ID_TAIL_BASE
UD_PW = 3120
UD_CHUNK = 1040
UD_CHUNKS = UD_PW // UD_CHUNK
UD_TAIL_BASE = UD_PW * NW
UD_TAIL = USER_NUM - UD_TAIL_BASE

_f32 = jnp.float32
_i32 = jnp.int32


def _sc_body(users_h, pos_h, negf_h, utab_h, itab_h, udeg_h, ideg_h,
             urows_o, prows_o, du_o, sq_o, udeg_o, pdeg_o, ss_o, mnu_o, mni_o,
             uidx_v, pidx_v, urows_v, prows_v, udeg_v, pdeg_v,
             nidx_v, nrows_v, dust_v, sqst_v, swp_v, dswp_v, part_v, sem):
    wid = lax.axis_index("s") * NC + lax.axis_index("c")
    ubase = wid * UPW

    pltpu.sync_copy(users_h.at[pl.ds(ubase, UPW)], uidx_v)
    pltpu.sync_copy(pos_h.at[pl.ds(ubase, UPW)], pidx_v)
    pltpu.async_copy(utab_h.at[uidx_v], urows_v, sem).wait()
    pltpu.async_copy(itab_h.at[pidx_v], prows_v, sem).wait()
    pltpu.async_copy(udeg_h.at[uidx_v], udeg_v, sem).wait()
    pltpu.async_copy(ideg_h.at[pidx_v], pdeg_v, sem).wait()
    pltpu.sync_copy(urows_v, urows_o.at[pl.ds(ubase, UPW)])
    pltpu.sync_copy(prows_v, prows_o.at[pl.ds(ubase, UPW)])
    pltpu.sync_copy(udeg_v, udeg_o.at[pl.ds(ubase, UPW)])
    pltpu.sync_copy(pdeg_v, pdeg_o.at[pl.ds(ubase, UPW)])

    lane = lax.iota(_i32, 16)
    nbase = ubase * NEG

    def chunk_body(c, carry):
        off = nbase + c * CH_ROWS
        pltpu.sync_copy(negf_h.at[pl.ds(off, CH_ROWS)], nidx_v)
        pltpu.async_copy(itab_h.at[nidx_v], nrows_v, sem).wait()

        def user_body(j, carry):
            urow = c * CH_USERS + j
            u0 = urows_v[urow, pl.ds(0, 16)]
            u1 = urows_v[urow, pl.ds(16, 16)]
            us = [u0[d] for d in range(16)] + [u1[d] for d in range(16)]
            jb = j * NEG

            def grp_body(g, carry):
                gb = jnp.minimum(g * 16, NEG - 16)
                rb = jb + gb
                rows = rb + lane
                accd = jnp.zeros((16,), _f32)
                accq = jnp.zeros((16,), _f32)
                for d2 in range(D):
                    col = plsc.load_gather(
                        nrows_v, [rows, jnp.full((16,), d2, _i32)])
                    accd = accd + us[d2] * col
                    accq = accq + col * col
                dust_v[pl.ds(rb, 16)] = accd
                sqst_v[pl.ds(rb, 16)] = accq
                return carry

            return lax.fori_loop(0, 13, grp_body, carry)

        carry = lax.fori_loop(0, CH_USERS, user_body, carry)
        pltpu.sync_copy(dust_v, du_o.at[pl.ds(off, CH_ROWS)])
        pltpu.sync_copy(sqst_v, sq_o.at[pl.ds(off, CH_ROWS)])
        return carry

    lax.fori_loop(0, NCH, chunk_body, 0)

    def sumsq_block(tab_h, base_row, nrows, acc):
        pltpu.sync_copy(tab_h.at[pl.ds(base_row, nrows)],
                        swp_v.at[pl.ds(0, nrows)])

        def rb_(r, acc):
            a = swp_v[r, pl.ds(0, 16)]
            b = swp_v[r, pl.ds(16, 16)]
            return acc + (a * a + b * b)

        return lax.fori_loop(0, nrows, rb_, acc)

    def sweep_table(tab_h, rows_pw, chunk_rows, n_chunks, acc):
        row0 = wid * rows_pw

        def cb(c, acc):
            return sumsq_block(tab_h, row0 + c * chunk_rows, chunk_rows, acc)

        return lax.fori_loop(0, n_chunks, cb, acc)

    acc = jnp.zeros((16,), _f32)
    acc = sweep_table(itab_h, IT_ROWS_PW, IT_SWEEP_ROWS, IT_CHUNKS, acc)
    acc = sweep_table(utab_h, UT_ROWS_PW, UT_SWEEP_ROWS, UT_CHUNKS, acc)
    it_tail = sumsq_block(itab_h, IT_TAIL_BASE, IT_TAIL, jnp.zeros((16,), _f32))
    acc = acc + jnp.where(wid == 0, it_tail, jnp.zeros((16,), _f32))
    ut_tail = sumsq_block(utab_h, UT_TAIL_BASE, UT_TAIL, jnp.zeros((16,), _f32))
    acc = acc + jnp.where(wid == 1, ut_tail, jnp.zeros((16,), _f32))
    part_v[pl.ds(0, 16)] = acc
    pltpu.sync_copy(part_v.at[pl.ds(0, 16)], ss_o.at[wid])

    def sweep_min(deg_h, base, nvecs, macc):
        pltpu.sync_copy(deg_h.at[pl.ds(base, nvecs * 16)],
                        dswp_v.at[pl.ds(0, nvecs * 16)])

        def vb(k, macc):
            return jnp.minimum(macc, dswp_v[pl.ds(k * 16, 16)])

        return lax.fori_loop(0, nvecs, vb, macc)

    big = jnp.full((16,), 3.0e38, _f32)

    def id_cb(c, macc):
        return sweep_min(ideg_h, wid * ID_PW + c * ID_CHUNK, ID_CHUNK // 16, macc)

    macc_i = lax.fori_loop(0, ID_CHUNKS, id_cb, big)
    macc_i = sweep_min(ideg_h, ID_TAIL_BASE, ID_TAIL // 16, macc_i)
    part_v[pl.ds(16, 16)] = macc_i
    pltpu.sync_copy(part_v.at[pl.ds(16, 16)], mni_o.at[wid])

    def ud_cb(c, macc):
        return sweep_min(udeg_h, wid * UD_PW + c * UD_CHUNK, UD_CHUNK // 16, macc)

    macc_u = lax.fori_loop(0, UD_CHUNKS, ud_cb, big)
    macc_u = sweep_min(udeg_h, UD_TAIL_BASE, UD_TAIL // 16, macc_u)
    part_v[pl.ds(32, 16)] = macc_u
    pltpu.sync_copy(part_v.at[pl.ds(32, 16)], mnu_o.at[wid])


def _sc_part(users, pos, negf, user_table, item_table, user_degree, item_degree):
    mesh = plsc.VectorSubcoreMesh(core_axis_name="c", subcore_axis_name="s",
                                  num_cores=NC, num_subcores=NS)
    kern = pl.kernel(
        _sc_body,
        out_type=(
            jax.ShapeDtypeStruct((B, D), _f32),
            jax.ShapeDtypeStruct((B, D), _f32),
            jax.ShapeDtypeStruct((B * NEG,), _f32),
            jax.ShapeDtypeStruct((B * NEG,), _f32),
            jax.ShapeDtypeStruct((B,), _f32),
            jax.ShapeDtypeStruct((B,), _f32),
            jax.ShapeDtypeStruct((NW, 16), _f32),
            jax.ShapeDtypeStruct((NW, 16), _f32),
            jax.ShapeDtypeStruct((NW, 16), _f32),
        ),
        mesh=mesh,
        compiler_params=pltpu.CompilerParams(needs_layout_passes=False,
                                             use_tc_tiling_on_sc=False),
        scratch_types=[
            pltpu.VMEM((UPW,), _i32),
            pltpu.VMEM((UPW,), _i32),
            pltpu.VMEM((UPW, D), _f32),
            pltpu.VMEM((UPW, D), _f32),
            pltpu.VMEM((UPW,), _f32),
            pltpu.VMEM((UPW,), _f32),
            pltpu.VMEM((CH_ROWS,), _i32),
            pltpu.VMEM((CH_ROWS, D), _f32),
            pltpu.VMEM((CH_ROWS,), _f32),
            pltpu.VMEM((CH_ROWS,), _f32),
            pltpu.VMEM((SWEEP_BUF_ROWS, D), _f32),
            pltpu.VMEM((UD_CHUNK,), _f32),
            pltpu.VMEM((48,), _f32),
            pltpu.SemaphoreType.DMA,
        ],
    )
    return kern(users, pos, negf, user_table, item_table, user_degree,
                item_degree)


def _loss_body(du_ref, sq_ref, u_ref, p_ref, ud_ref, pd_ref, ss_ref,
               mnu_ref, mni_ref, tot_ref, l1_ref, l2_ref, reg_ref):
    u = u_ref[...]
    p = p_ref[...]
    squ = jnp.sum(u * u, axis=1, keepdims=True)
    dup = jnp.sum(u * p, axis=1, keepdims=True)
    sqp = jnp.sum(p * p, axis=1, keepdims=True)
    cu = jnp.maximum(jnp.sqrt(squ), 1e-12)
    cp = jnp.maximum(jnp.sqrt(sqp), 1e-12)
    du = du_ref[...]
    sq = sq_ref[...]
    cn = jnp.maximum(jnp.sqrt(sq), 1e-12)
    pos_sc = dup / cu
    npos_sc = dup / (cu * cp)
    neg_sc = du / cu
    nneg_sc = du / (cu * cn)
    udeg = ud_ref[...]
    pdeg = pd_ref[...]
    upw = jnp.log(udeg * 1000.0)
    pw = jnp.log(pdeg * 1000.0)
    minu = jnp.min(mnu_ref[...])
    mini = jnp.min(mni_ref[...])
    npw = pw / (-jnp.log(mini * 1000.0 + 1e-7))
    nuw = upw / (-jnp.log(minu * 1000.0 + 1e-7))

    ep1 = jnp.exp((pos_sc + pw + upw) / MARGIN1)
    en1 = jnp.exp(neg_sc / MARGIN1)
    ns1 = jnp.mean(en1, axis=1, keepdims=True)
    d1 = NEG * ns1 + ep1 + 1e-7
    l1 = -jnp.mean(jnp.log(ep1 / d1))

    ep2 = jnp.exp((npos_sc + npw + nuw) / MARGIN2)
    en2 = jnp.exp(nneg_sc / MARGIN2)
    ns2 = jnp.mean(en2, axis=1, keepdims=True)
    d2 = NEG * ns2 + ep2 + 1e-7
    l2 = -jnp.mean(jnp.log(ep2 / d2))

    ss = jnp.sum(ss_ref[...])
    reg = GAMMA * ss / 2.0
    l1w = WEIGHT * l1
    tot_ref[...] = jnp.reshape(l1w + l2 + reg, (1, 1))
    l1_ref[...] = jnp.reshape(l1w, (1, 1))
    l2_ref[...] = jnp.reshape(l2, (1, 1))
    reg_ref[...] = jnp.reshape(reg, (1, 1))


def kernel(users, pos_items, neg_items, user_table, item_table, user_degree,
           item_degree):
    users = users.astype(_i32)
    pos = pos_items.astype(_i32)
    negf = neg_items.astype(_i32).reshape(-1)
    (urows, prows, du, sq, udeg, pdeg, ss, mnu, mni) = _sc_part(
        users, pos, negf, user_table, item_table, user_degree, item_degree)
    du2 = du.reshape(B, NEG)
    sq2 = sq.reshape(B, NEG)
    out = pl.pallas_call(
        _loss_body,
        out_shape=[jax.ShapeDtypeStruct((1, 1), _f32)] * 4,
    )(du2, sq2, urows, prows, udeg.reshape(B, 1), pdeg.reshape(B, 1),
      ss, mnu, mni)
    tot, l1w, l2, reg = out
    return (tot[0, 0], l1w[0, 0], l2[0, 0], reg[0, 0])


# speedup vs baseline: 1.2546x; 1.2546x over previous
import functools

import jax
import jax.numpy as jnp
from jax import lax
from jax.experimental import pallas as pl
from jax.experimental.pallas import tpu as pltpu
from jax.experimental.pallas import tpu_sc as plsc

B = 4096
NEG = 200
D = 32
USER_NUM = 100000
ITEM_NUM = 1000000
WEIGHT = 0.5
MARGIN1 = 4.0
MARGIN2 = 0.5
GAMMA = 1e-4

NC = 2
NS = 16
NW = NC * NS
UPW = B // NW
CH_USERS = 4
CH_ROWS = CH_USERS * NEG
NCH = UPW // CH_USERS

_f32 = jnp.float32
_i32 = jnp.int32


def _sc_body(users_h, pos_h, negf_h, utab_h, itab_h, udeg_h, ideg_h,
             urows_o, prows_o, du_o, sq_o, udeg_o, pdeg_o,
             uidx_v, pidx_v, urows_v, prows_v, udeg_v, pdeg_v, nidx_v,
             nrows_a, nrows_b, dust_a, dust_b, sqst_a, sqst_b,
             gsem_a, gsem_b, osem_a, osem_b, sem):
    wid = lax.axis_index("s") * NC + lax.axis_index("c")
    ubase = wid * UPW
    nbase = ubase * NEG

    pltpu.sync_copy(users_h.at[pl.ds(ubase, UPW)], uidx_v)
    pltpu.sync_copy(pos_h.at[pl.ds(ubase, UPW)], pidx_v)
    pltpu.sync_copy(negf_h.at[pl.ds(nbase, UPW * NEG)], nidx_v)
    pltpu.async_copy(utab_h.at[uidx_v], urows_v, sem).wait()
    pltpu.async_copy(itab_h.at[pidx_v], prows_v, sem).wait()
    pltpu.async_copy(udeg_h.at[uidx_v], udeg_v, sem).wait()
    pltpu.async_copy(ideg_h.at[pidx_v], pdeg_v, sem).wait()
    pltpu.sync_copy(urows_v, urows_o.at[pl.ds(ubase, UPW)])
    pltpu.sync_copy(prows_v, prows_o.at[pl.ds(ubase, UPW)])
    pltpu.sync_copy(udeg_v, udeg_o.at[pl.ds(ubase, UPW)])
    pltpu.sync_copy(pdeg_v, pdeg_o.at[pl.ds(ubase, UPW)])

    lane = lax.iota(_i32, 16)

    def start_gather(c, nrows_v, gsem):
        pltpu.make_async_copy(
            itab_h.at[nidx_v.at[pl.ds(c * CH_ROWS, CH_ROWS)]],
            nrows_v, gsem).start()

    def wait_gather(nrows_v, gsem):
        pltpu.make_async_copy(
            itab_h.at[nidx_v.at[pl.ds(0, CH_ROWS)]], nrows_v, gsem).wait()

    def start_out(c, dust_v, sqst_v, osem):
        off = nbase + c * CH_ROWS
        pltpu.make_async_copy(dust_v, du_o.at[pl.ds(off, CH_ROWS)],
                              osem).start()
        pltpu.make_async_copy(sqst_v, sq_o.at[pl.ds(off, CH_ROWS)],
                              osem).start()

    def wait_out(dust_v, sqst_v, osem):
        pltpu.make_async_copy(dust_v, du_o.at[pl.ds(nbase, CH_ROWS)],
                              osem).wait()
        pltpu.make_async_copy(sqst_v, sq_o.at[pl.ds(nbase, CH_ROWS)],
                              osem).wait()

    def compute_chunk(c, nrows_v, dust_v, sqst_v):
        def user_body(j, carry):
            urow = c * CH_USERS + j
            u0 = urows_v[urow, pl.ds(0, 16)]
            u1 = urows_v[urow, pl.ds(16, 16)]
            us = [u0[d] for d in range(16)] + [u1[d] for d in range(16)]
            jb = j * NEG

            def grp_body(g, carry):
                gb = jnp.minimum(g * 16, NEG - 16)
                rb = jb + gb
                rows = rb + lane
                accd = jnp.zeros((16,), _f32)
                accq = jnp.zeros((16,), _f32)
                for d2 in range(D):
                    col = plsc.load_gather(
                        nrows_v, [rows, jnp.full((16,), d2, _i32)])
                    accd = accd + us[d2] * col
                    accq = accq + col * col
                dust_v[pl.ds(rb, 16)] = accd
                sqst_v[pl.ds(rb, 16)] = accq
                return carry

            return lax.fori_loop(0, 13, grp_body, carry)

        lax.fori_loop(0, CH_USERS, user_body, 0)

    start_gather(0, nrows_a, gsem_a)

    def pair_body(t, carry):
        c0 = 2 * t
        c1 = c0 + 1
        start_gather(c1, nrows_b, gsem_b)
        wait_gather(nrows_a, gsem_a)

        @pl.when(t > 0)
        def _():
            wait_out(dust_a, sqst_a, osem_a)

        compute_chunk(c0, nrows_a, dust_a, sqst_a)
        start_out(c0, dust_a, sqst_a, osem_a)

        @pl.when(t < NCH // 2 - 1)
        def _():
            start_gather(c0 + 2, nrows_a, gsem_a)

        wait_gather(nrows_b, gsem_b)

        @pl.when(t > 0)
        def _():
            wait_out(dust_b, sqst_b, osem_b)

        compute_chunk(c1, nrows_b, dust_b, sqst_b)
        start_out(c1, dust_b, sqst_b, osem_b)
        return carry

    lax.fori_loop(0, NCH // 2, pair_body, 0)
    wait_out(dust_a, sqst_a, osem_a)
    wait_out(dust_b, sqst_b, osem_b)


def _sc_part(users, pos, negf, user_table, item_table, user_degree,
             item_degree):
    mesh = plsc.VectorSubcoreMesh(core_axis_name="c", subcore_axis_name="s",
                                  num_cores=NC, num_subcores=NS)
    kern = pl.kernel(
        _sc_body,
        out_type=(
            jax.ShapeDtypeStruct((B, D), _f32),
            jax.ShapeDtypeStruct((B, D), _f32),
            jax.ShapeDtypeStruct((B * NEG,), _f32),
            jax.ShapeDtypeStruct((B * NEG,), _f32),
            jax.ShapeDtypeStruct((B,), _f32),
            jax.ShapeDtypeStruct((B,), _f32),
        ),
        mesh=mesh,
        compiler_params=pltpu.CompilerParams(needs_layout_passes=False,
                                             use_tc_tiling_on_sc=False),
        scratch_types=[
            pltpu.VMEM((UPW,), _i32),
            pltpu.VMEM((UPW,), _i32),
            pltpu.VMEM((UPW, D), _f32),
            pltpu.VMEM((UPW, D), _f32),
            pltpu.VMEM((UPW,), _f32),
            pltpu.VMEM((UPW,), _f32),
            pltpu.VMEM((UPW * NEG,), _i32),
            pltpu.VMEM((CH_ROWS, D), _f32),
            pltpu.VMEM((CH_ROWS, D), _f32),
            pltpu.VMEM((CH_ROWS,), _f32),
            pltpu.VMEM((CH_ROWS,), _f32),
            pltpu.VMEM((CH_ROWS,), _f32),
            pltpu.VMEM((CH_ROWS,), _f32),
            pltpu.SemaphoreType.DMA,
            pltpu.SemaphoreType.DMA,
            pltpu.SemaphoreType.DMA,
            pltpu.SemaphoreType.DMA,
            pltpu.SemaphoreType.DMA,
        ],
    )
    return kern(users, pos, negf, user_table, item_table, user_degree,
                item_degree)


IT_BLK = 8000
UT_BLK = 800
RED_GRID = ITEM_NUM // IT_BLK


def _red_body(it_ref, ut_ref, id_ref, ud_ref, ss_ref, mnu_ref, mni_ref):
    i = pl.program_id(0)
    it = it_ref[...]
    ut = ut_ref[...]
    s = jnp.sum(it * it) + jnp.sum(ut * ut)
    mi = jnp.min(id_ref[...])
    mu = jnp.min(ud_ref[...])

    @pl.when(i == 0)
    def _():
        ss_ref[0, 0] = s
        mni_ref[0, 0] = mi
        mnu_ref[0, 0] = mu

    @pl.when(i > 0)
    def _():
        ss_ref[0, 0] += s
        mni_ref[0, 0] = jnp.minimum(mni_ref[0, 0], mi)
        mnu_ref[0, 0] = jnp.minimum(mnu_ref[0, 0], mu)


def _tc_reduce(user_table, item_table, user_degree, item_degree):
    id2 = item_degree.reshape(RED_GRID, 1, ITEM_NUM // RED_GRID)
    ud2 = user_degree.reshape(RED_GRID, 1, USER_NUM // RED_GRID)
    return pl.pallas_call(
        _red_body,
        grid=(RED_GRID,),
        in_specs=[
            pl.BlockSpec((IT_BLK, D), lambda i: (i, 0)),
            pl.BlockSpec((UT_BLK, D), lambda i: (i, 0)),
            pl.BlockSpec((1, 1, ITEM_NUM // RED_GRID), lambda i: (i, 0, 0)),
            pl.BlockSpec((1, 1, USER_NUM // RED_GRID), lambda i: (i, 0, 0)),
        ],
        out_specs=[
            pl.BlockSpec((1, 1), lambda i: (0, 0),
                         memory_space=pltpu.SMEM),
            pl.BlockSpec((1, 1), lambda i: (0, 0),
                         memory_space=pltpu.SMEM),
            pl.BlockSpec((1, 1), lambda i: (0, 0),
                         memory_space=pltpu.SMEM),
        ],
        out_shape=[jax.ShapeDtypeStruct((1, 1), _f32)] * 3,
    )(item_table, user_table, id2, ud2)


def _loss_body(du_ref, sq_ref, u_ref, p_ref, ud_ref, pd_ref, ss_ref,
               mnu_ref, mni_ref, tot_ref, l1_ref, l2_ref, reg_ref):
    u = u_ref[...]
    p = p_ref[...]
    squ = jnp.sum(u * u, axis=1, keepdims=True)
    dup = jnp.sum(u * p, axis=1, keepdims=True)
    sqp = jnp.sum(p * p, axis=1, keepdims=True)
    cu = jnp.maximum(jnp.sqrt(squ), 1e-12)
    cp = jnp.maximum(jnp.sqrt(sqp), 1e-12)
    du = du_ref[...]
    sq = sq_ref[...]
    cn = jnp.maximum(jnp.sqrt(sq), 1e-12)
    pos_sc = dup / cu
    npos_sc = dup / (cu * cp)
    neg_sc = du / cu
    nneg_sc = du / (cu * cn)
    udeg = ud_ref[...]
    pdeg = pd_ref[...]
    upw = jnp.log(udeg * 1000.0)
    pw = jnp.log(pdeg * 1000.0)
    minu = mnu_ref[0, 0]
    mini = mni_ref[0, 0]
    npw = pw / (-jnp.log(mini * 1000.0 + 1e-7))
    nuw = upw / (-jnp.log(minu * 1000.0 + 1e-7))

    ep1 = jnp.exp((pos_sc + pw + upw) / MARGIN1)
    en1 = jnp.exp(neg_sc / MARGIN1)
    ns1 = jnp.mean(en1, axis=1, keepdims=True)
    d1 = NEG * ns1 + ep1 + 1e-7
    l1 = -jnp.mean(jnp.log(ep1 / d1))

    ep2 = jnp.exp((npos_sc + npw + nuw) / MARGIN2)
    en2 = jnp.exp(nneg_sc / MARGIN2)
    ns2 = jnp.mean(en2, axis=1, keepdims=True)
    d2 = NEG * ns2 + ep2 + 1e-7
    l2 = -jnp.mean(jnp.log(ep2 / d2))

    reg = GAMMA * ss_ref[0, 0] / 2.0
    l1w = WEIGHT * l1
    tot_ref[0, 0] = l1w + l2 + reg
    l1_ref[0, 0] = l1w
    l2_ref[0, 0] = l2
    reg_ref[0, 0] = reg


def kernel(users, pos_items, neg_items, user_table, item_table, user_degree,
           item_degree):
    users = users.astype(_i32)
    pos = pos_items.astype(_i32)
    negf = neg_items.astype(_i32).reshape(-1)
    (urows, prows, du, sq, udeg, pdeg) = _sc_part(
        users, pos, negf, user_table, item_table, user_degree, item_degree)
    ss, mnu, mni = _tc_reduce(user_table, item_table, user_degree,
                              item_degree)
    du2 = du.reshape(B, NEG)
    sq2 = sq.reshape(B, NEG)
    out = pl.pallas_call(
        _loss_body,
        in_specs=[
            pl.BlockSpec((B, NEG), lambda: (0, 0)),
            pl.BlockSpec((B, NEG), lambda: (0, 0)),
            pl.BlockSpec((B, D), lambda: (0, 0)),
            pl.BlockSpec((B, D), lambda: (0, 0)),
            pl.BlockSpec((B, 1), lambda: (0, 0)),
            pl.BlockSpec((B, 1), lambda: (0, 0)),
            pl.BlockSpec((1, 1), lambda: (0, 0), memory_space=pltpu.SMEM),
            pl.BlockSpec((1, 1), lambda: (0, 0), memory_space=pltpu.SMEM),
            pl.BlockSpec((1, 1), lambda: (0, 0), memory_space=pltpu.SMEM),
        ],
        out_shape=[jax.ShapeDtypeStruct((1, 1), _f32)] * 4,
        out_specs=[pl.BlockSpec((1, 1), lambda: (0, 0),
                                memory_space=pltpu.SMEM)] * 4,
    )(du2, sq2, urows, prows, udeg.reshape(B, 1), pdeg.reshape(B, 1),
      ss, mnu, mni)
    tot, l1w, l2, reg = out
    return (tot[0, 0], l1w[0, 0], l2[0, 0], reg[0, 0])


# speedup vs baseline: 1.3658x; 1.0886x over previous
import functools

import jax
import jax.numpy as jnp
from jax import lax
from jax.experimental import pallas as pl
from jax.experimental.pallas import tpu as pltpu
from jax.experimental.pallas import tpu_sc as plsc

B = 4096
NEG = 200
D = 32
USER_NUM = 100000
ITEM_NUM = 1000000
WEIGHT = 0.5
MARGIN1 = 4.0
MARGIN2 = 0.5
GAMMA = 1e-4

NC = 2
NS = 16
NW = NC * NS
UPW = B // NW
CH_USERS = 4
CH_ROWS = CH_USERS * NEG
NCH = UPW // CH_USERS

IT_PW = 31248
IT_CH = 744
IT_NP = IT_PW // IT_CH // 2
IT_TAIL_BASE = IT_PW * NW
IT_TAIL = ITEM_NUM - IT_TAIL_BASE
UT_PW = 3120
UT_CH = 312
UT_NP = UT_PW // UT_CH // 2
UT_TAIL_BASE = UT_PW * NW
UT_TAIL = USER_NUM - UT_TAIL_BASE

_f32 = jnp.float32
_i32 = jnp.int32


def _sc_body(users_h, pos_h, negf_h, utab_h, itab_h, udeg_h, ideg_h,
             urows_o, prows_o, du_o, sq_o, udeg_o, pdeg_o, ss_o,
             uidx_v, pidx_v, urows_v, prows_v, udeg_v, pdeg_v, nidx_v,
             nrows_a, nrows_b, dust_a, dust_b, sqst_a, sqst_b, part_v,
             gsem_a, gsem_b, osem_a, osem_b, sem):
    wid = lax.axis_index("s") * NC + lax.axis_index("c")
    ubase = wid * UPW
    nbase = ubase * NEG

    pltpu.sync_copy(users_h.at[pl.ds(ubase, UPW)], uidx_v)
    pltpu.sync_copy(pos_h.at[pl.ds(ubase, UPW)], pidx_v)
    pltpu.sync_copy(negf_h.at[pl.ds(nbase, UPW * NEG)], nidx_v)
    pltpu.async_copy(utab_h.at[uidx_v], urows_v, sem).wait()
    pltpu.async_copy(itab_h.at[pidx_v], prows_v, sem).wait()
    pltpu.async_copy(udeg_h.at[uidx_v], udeg_v, sem).wait()
    pltpu.async_copy(ideg_h.at[pidx_v], pdeg_v, sem).wait()
    pltpu.sync_copy(urows_v, urows_o.at[pl.ds(ubase, UPW)])
    pltpu.sync_copy(prows_v, prows_o.at[pl.ds(ubase, UPW)])
    pltpu.sync_copy(udeg_v, udeg_o.at[pl.ds(ubase, UPW)])
    pltpu.sync_copy(pdeg_v, pdeg_o.at[pl.ds(ubase, UPW)])

    lane = lax.iota(_i32, 16)

    def start_gather(c, nrows_v, gsem):
        pltpu.make_async_copy(
            itab_h.at[nidx_v.at[pl.ds(c * CH_ROWS, CH_ROWS)]],
            nrows_v, gsem).start()

    def wait_gather(nrows_v, gsem):
        pltpu.make_async_copy(
            itab_h.at[nidx_v.at[pl.ds(0, CH_ROWS)]], nrows_v, gsem).wait()

    def start_out(c, dust_v, sqst_v, osem):
        off = nbase + c * CH_ROWS
        pltpu.make_async_copy(dust_v, du_o.at[pl.ds(off, CH_ROWS)],
                              osem).start()
        pltpu.make_async_copy(sqst_v, sq_o.at[pl.ds(off, CH_ROWS)],
                              osem).start()

    def wait_out(dust_v, sqst_v, osem):
        pltpu.make_async_copy(dust_v, du_o.at[pl.ds(nbase, CH_ROWS)],
                              osem).wait()
        pltpu.make_async_copy(sqst_v, sq_o.at[pl.ds(nbase, CH_ROWS)],
                              osem).wait()

    def compute_chunk(c, nrows_v, dust_v, sqst_v):
        def user_body(j, carry):
            urow = c * CH_USERS + j
            u0 = urows_v[urow, pl.ds(0, 16)]
            u1 = urows_v[urow, pl.ds(16, 16)]
            us = [u0[d] for d in range(16)] + [u1[d] for d in range(16)]
            jb = j * NEG

            def grp_body(g, carry):
                gb = jnp.minimum(g * 16, NEG - 16)
                rb = jb + gb
                rows = rb + lane
                accd = jnp.zeros((16,), _f32)
                accq = jnp.zeros((16,), _f32)
                for d2 in range(D):
                    col = plsc.load_gather(
                        nrows_v, [rows, jnp.full((16,), d2, _i32)])
                    accd = accd + us[d2] * col
                    accq = accq + col * col
                dust_v[pl.ds(rb, 16)] = accd
                sqst_v[pl.ds(rb, 16)] = accq
                return carry

            return lax.fori_loop(0, 13, grp_body, carry)

        lax.fori_loop(0, CH_USERS, user_body, 0)

    start_gather(0, nrows_a, gsem_a)

    def pair_body(t, carry):
        c0 = 2 * t
        c1 = c0 + 1
        start_gather(c1, nrows_b, gsem_b)
        wait_gather(nrows_a, gsem_a)

        @pl.when(t > 0)
        def _():
            wait_out(dust_a, sqst_a, osem_a)

        compute_chunk(c0, nrows_a, dust_a, sqst_a)
        start_out(c0, dust_a, sqst_a, osem_a)

        @pl.when(t < NCH // 2 - 1)
        def _():
            start_gather(c0 + 2, nrows_a, gsem_a)

        wait_gather(nrows_b, gsem_b)

        @pl.when(t > 0)
        def _():
            wait_out(dust_b, sqst_b, osem_b)

        compute_chunk(c1, nrows_b, dust_b, sqst_b)
        start_out(c1, dust_b, sqst_b, osem_b)
        return carry

    lax.fori_loop(0, NCH // 2, pair_body, 0)
    wait_out(dust_a, sqst_a, osem_a)
    wait_out(dust_b, sqst_b, osem_b)

    def accum_rows(buf, nrows, acc):
        def rb_(k, acc):
            for r8 in range(8):
                a = buf[k * 8 + r8, pl.ds(0, 16)]
                b = buf[k * 8 + r8, pl.ds(16, 16)]
                acc = acc + a * a
                acc = acc + b * b
            return acc

        return lax.fori_loop(0, nrows // 8, rb_, acc)

    def start_sweep(tab_h, row, chunk, buf, gsem):
        pltpu.make_async_copy(tab_h.at[pl.ds(row, chunk)],
                              buf.at[pl.ds(0, chunk)], gsem).start()

    def wait_sweep(tab_h, chunk, buf, gsem):
        pltpu.make_async_copy(tab_h.at[pl.ds(0, chunk)],
                              buf.at[pl.ds(0, chunk)], gsem).wait()

    def sweep_pair(tab_h, row0, chunk, npairs, acc):
        start_sweep(tab_h, row0, chunk, nrows_a, gsem_a)

        def pb(t, acc):
            start_sweep(tab_h, row0 + (2 * t + 1) * chunk, chunk, nrows_b,
                        gsem_b)
            wait_sweep(tab_h, chunk, nrows_a, gsem_a)
            acc = accum_rows(nrows_a, chunk, acc)

            @pl.when(t < npairs - 1)
            def _():
                start_sweep(tab_h, row0 + (2 * t + 2) * chunk, chunk,
                            nrows_a, gsem_a)

            wait_sweep(tab_h, chunk, nrows_b, gsem_b)
            acc = accum_rows(nrows_b, chunk, acc)
            return acc

        return lax.fori_loop(0, npairs, pb, acc)

    acc = jnp.zeros((16,), _f32)
    acc = sweep_pair(itab_h, wid * IT_PW, IT_CH, IT_NP, acc)
    acc = sweep_pair(utab_h, wid * UT_PW, UT_CH, UT_NP, acc)
    pltpu.sync_copy(itab_h.at[pl.ds(IT_TAIL_BASE, IT_TAIL)],
                    nrows_a.at[pl.ds(0, IT_TAIL)])
    t_it = accum_rows(nrows_a, IT_TAIL, jnp.zeros((16,), _f32))
    acc = acc + jnp.where(wid == 0, t_it, jnp.zeros((16,), _f32))
    pltpu.sync_copy(utab_h.at[pl.ds(UT_TAIL_BASE, UT_TAIL)],
                    nrows_a.at[pl.ds(0, UT_TAIL)])
    t_ut = accum_rows(nrows_a, UT_TAIL, jnp.zeros((16,), _f32))
    acc = acc + jnp.where(wid == 1, t_ut, jnp.zeros((16,), _f32))
    part_v[pl.ds(0, 16)] = acc
    pltpu.sync_copy(part_v, ss_o.at[wid])


def _sc_part(users, pos, negf, user_table, item_table, user_degree,
             item_degree):
    mesh = plsc.VectorSubcoreMesh(core_axis_name="c", subcore_axis_name="s",
                                  num_cores=NC, num_subcores=NS)
    kern = pl.kernel(
        _sc_body,
        out_type=(
            jax.ShapeDtypeStruct((B, D), _f32),
            jax.ShapeDtypeStruct((B, D), _f32),
            jax.ShapeDtypeStruct((B * NEG,), _f32),
            jax.ShapeDtypeStruct((B * NEG,), _f32),
            jax.ShapeDtypeStruct((B,), _f32),
            jax.ShapeDtypeStruct((B,), _f32),
            jax.ShapeDtypeStruct((NW, 16), _f32),
        ),
        mesh=mesh,
        compiler_params=pltpu.CompilerParams(needs_layout_passes=False,
                                             use_tc_tiling_on_sc=False),
        scratch_types=[
            pltpu.VMEM((UPW,), _i32),
            pltpu.VMEM((UPW,), _i32),
            pltpu.VMEM((UPW, D), _f32),
            pltpu.VMEM((UPW, D), _f32),
            pltpu.VMEM((UPW,), _f32),
            pltpu.VMEM((UPW,), _f32),
            pltpu.VMEM((UPW * NEG,), _i32),
            pltpu.VMEM((CH_ROWS, D), _f32),
            pltpu.VMEM((CH_ROWS, D), _f32),
            pltpu.VMEM((CH_ROWS,), _f32),
            pltpu.VMEM((CH_ROWS,), _f32),
            pltpu.VMEM((CH_ROWS,), _f32),
            pltpu.VMEM((CH_ROWS,), _f32),
            pltpu.VMEM((16,), _f32),
            pltpu.SemaphoreType.DMA,
            pltpu.SemaphoreType.DMA,
            pltpu.SemaphoreType.DMA,
            pltpu.SemaphoreType.DMA,
            pltpu.SemaphoreType.DMA,
        ],
    )
    return kern(users, pos, negf, user_table, item_table, user_degree,
                item_degree)


RED_GRID = 125


def _min_body(id_ref, ud_ref, mnu_ref, mni_ref):
    i = pl.program_id(0)
    mi = jnp.min(id_ref[...])
    mu = jnp.min(ud_ref[...])

    @pl.when(i == 0)
    def _():
        mni_ref[0, 0] = mi
        mnu_ref[0, 0] = mu

    @pl.when(i > 0)
    def _():
        mni_ref[0, 0] = jnp.minimum(mni_ref[0, 0], mi)
        mnu_ref[0, 0] = jnp.minimum(mnu_ref[0, 0], mu)


def _tc_minred(user_degree, item_degree):
    id2 = item_degree.reshape(RED_GRID, 1, ITEM_NUM // RED_GRID)
    ud2 = user_degree.reshape(RED_GRID, 1, USER_NUM // RED_GRID)
    return pl.pallas_call(
        _min_body,
        grid=(RED_GRID,),
        in_specs=[
            pl.BlockSpec((1, 1, ITEM_NUM // RED_GRID), lambda i: (i, 0, 0)),
            pl.BlockSpec((1, 1, USER_NUM // RED_GRID), lambda i: (i, 0, 0)),
        ],
        out_specs=[
            pl.BlockSpec((1, 1), lambda i: (0, 0), memory_space=pltpu.SMEM),
            pl.BlockSpec((1, 1), lambda i: (0, 0), memory_space=pltpu.SMEM),
        ],
        out_shape=[jax.ShapeDtypeStruct((1, 1), _f32)] * 2,
    )(id2, ud2)


def _loss_body(du_ref, sq_ref, u_ref, p_ref, ud_ref, pd_ref, ss_ref,
               mnu_ref, mni_ref, tot_ref, l1_ref, l2_ref, reg_ref):
    u = u_ref[...]
    p = p_ref[...]
    squ = jnp.sum(u * u, axis=1, keepdims=True)
    dup = jnp.sum(u * p, axis=1, keepdims=True)
    sqp = jnp.sum(p * p, axis=1, keepdims=True)
    cu = jnp.maximum(jnp.sqrt(squ), 1e-12)
    cp = jnp.maximum(jnp.sqrt(sqp), 1e-12)
    du = du_ref[...]
    sq = sq_ref[...]
    cn = jnp.maximum(jnp.sqrt(sq), 1e-12)
    pos_sc = dup / cu
    npos_sc = dup / (cu * cp)
    neg_sc = du / cu
    nneg_sc = du / (cu * cn)
    udeg = ud_ref[...]
    pdeg = pd_ref[...]
    upw = jnp.log(udeg * 1000.0)
    pw = jnp.log(pdeg * 1000.0)
    minu = mnu_ref[0, 0]
    mini = mni_ref[0, 0]
    npw = pw / (-jnp.log(mini * 1000.0 + 1e-7))
    nuw = upw / (-jnp.log(minu * 1000.0 + 1e-7))

    ep1 = jnp.exp((pos_sc + pw + upw) / MARGIN1)
    en1 = jnp.exp(neg_sc / MARGIN1)
    ns1 = jnp.mean(en1, axis=1, keepdims=True)
    d1 = NEG * ns1 + ep1 + 1e-7
    l1 = -jnp.mean(jnp.log(ep1 / d1))

    ep2 = jnp.exp((npos_sc + npw + nuw) / MARGIN2)
    en2 = jnp.exp(nneg_sc / MARGIN2)
    ns2 = jnp.mean(en2, axis=1, keepdims=True)
    d2 = NEG * ns2 + ep2 + 1e-7
    l2 = -jnp.mean(jnp.log(ep2 / d2))

    reg = GAMMA * jnp.sum(ss_ref[...]) / 2.0
    l1w = WEIGHT * l1
    tot_ref[0, 0] = l1w + l2 + reg
    l1_ref[0, 0] = l1w
    l2_ref[0, 0] = l2
    reg_ref[0, 0] = reg


def kernel(users, pos_items, neg_items, user_table, item_table, user_degree,
           item_degree):
    users = users.astype(_i32)
    pos = pos_items.astype(_i32)
    negf = neg_items.astype(_i32).reshape(-1)
    (urows, prows, du, sq, udeg, pdeg, ss) = _sc_part(
        users, pos, negf, user_table, item_table, user_degree, item_degree)
    mnu, mni = _tc_minred(user_degree, item_degree)
    du2 = du.reshape(B, NEG)
    sq2 = sq.reshape(B, NEG)
    out = pl.pallas_call(
        _loss_body,
        in_specs=[
            pl.BlockSpec((B, NEG), lambda: (0, 0)),
            pl.BlockSpec((B, NEG), lambda: (0, 0)),
            pl.BlockSpec((B, D), lambda: (0, 0)),
            pl.BlockSpec((B, D), lambda: (0, 0)),
            pl.BlockSpec((B, 1), lambda: (0, 0)),
            pl.BlockSpec((B, 1), lambda: (0, 0)),
            pl.BlockSpec((NW, 16), lambda: (0, 0)),
            pl.BlockSpec((1, 1), lambda: (0, 0), memory_space=pltpu.SMEM),
            pl.BlockSpec((1, 1), lambda: (0, 0), memory_space=pltpu.SMEM),
        ],
        out_shape=[jax.ShapeDtypeStruct((1, 1), _f32)] * 4,
        out_specs=[pl.BlockSpec((1, 1), lambda: (0, 0),
                                memory_space=pltpu.SMEM)] * 4,
    )(du2, sq2, urows, prows, udeg.reshape(B, 1), pdeg.reshape(B, 1),
      ss, mnu, mni)
    tot, l1w, l2, reg = out
    return (tot[0, 0], l1w[0, 0], l2[0, 0], reg[0, 0])


# speedup vs baseline: 1.8306x; 1.3403x over previous
import functools

import jax
import jax.numpy as jnp
from jax import lax
from jax.experimental import pallas as pl
from jax.experimental.pallas import tpu as pltpu
from jax.experimental.pallas import tpu_sc as plsc

B = 4096
NEG = 200
D = 32
USER_NUM = 100000
ITEM_NUM = 1000000
WEIGHT = 0.5
MARGIN1 = 4.0
MARGIN2 = 0.5
GAMMA = 1e-4

NC = 2
NS = 16
NW = NC * NS
UPW = B // NW
CH_USERS = 4
CH_ROWS = CH_USERS * NEG
NCH = UPW // CH_USERS

IT_PW = 31248
IT_CH = 744
IT_NP = IT_PW // IT_CH // 2
IT_TAIL_BASE = IT_PW * NW
IT_TAIL = ITEM_NUM - IT_TAIL_BASE
UT_PW = 3120
UT_CH = 312
UT_NP = UT_PW // UT_CH // 2
UT_TAIL_BASE = UT_PW * NW
UT_TAIL = USER_NUM - UT_TAIL_BASE

_f32 = jnp.float32
_i32 = jnp.int32


def _sc_body(users_h, pos_h, negf_h, utab_h, itab_h, udeg_h, ideg_h,
             urows_o, prows_o, du_o, sq_o, udeg_o, pdeg_o, ss_o,
             uidx_v, pidx_v, urows_v, prows_v, udeg_v, pdeg_v, nidx_v,
             nrows_a, nrows_b, dust_a, dust_b, sqst_a, sqst_b, part_v,
             gsem_a, gsem_b, osem_a, osem_b, sem):
    wid = lax.axis_index("s") * NC + lax.axis_index("c")
    ubase = wid * UPW
    nbase = ubase * NEG

    pltpu.sync_copy(users_h.at[pl.ds(ubase, UPW)], uidx_v)
    pltpu.sync_copy(pos_h.at[pl.ds(ubase, UPW)], pidx_v)
    pltpu.sync_copy(negf_h.at[pl.ds(nbase, UPW * NEG)], nidx_v)
    pltpu.async_copy(utab_h.at[uidx_v], urows_v, sem).wait()
    pltpu.async_copy(itab_h.at[pidx_v], prows_v, sem).wait()
    pltpu.async_copy(udeg_h.at[uidx_v], udeg_v, sem).wait()
    pltpu.async_copy(ideg_h.at[pidx_v], pdeg_v, sem).wait()
    pltpu.sync_copy(urows_v, urows_o.at[pl.ds(ubase, UPW)])
    pltpu.sync_copy(prows_v, prows_o.at[pl.ds(ubase, UPW)])
    pltpu.sync_copy(udeg_v, udeg_o.at[pl.ds(ubase, UPW)])
    pltpu.sync_copy(pdeg_v, pdeg_o.at[pl.ds(ubase, UPW)])

    lane = lax.iota(_i32, 16)

    def start_gather(c, nrows_v, gsem):
        pltpu.make_async_copy(
            itab_h.at[nidx_v.at[pl.ds(c * CH_ROWS, CH_ROWS)]],
            nrows_v, gsem).start()

    def wait_gather(nrows_v, gsem):
        pltpu.make_async_copy(
            itab_h.at[nidx_v.at[pl.ds(0, CH_ROWS)]], nrows_v, gsem).wait()

    def start_out(c, dust_v, sqst_v, osem):
        off = nbase + c * CH_ROWS
        pltpu.make_async_copy(dust_v, du_o.at[pl.ds(off, CH_ROWS)],
                              osem).start()
        pltpu.make_async_copy(sqst_v, sq_o.at[pl.ds(off, CH_ROWS)],
                              osem).start()

    def wait_out(dust_v, sqst_v, osem):
        pltpu.make_async_copy(dust_v, du_o.at[pl.ds(nbase, CH_ROWS)],
                              osem).wait()
        pltpu.make_async_copy(sqst_v, sq_o.at[pl.ds(nbase, CH_ROWS)],
                              osem).wait()

    def _perm(v, idx):
        return lax.gather(
            v, idx[:, None],
            lax.GatherDimensionNumbers(offset_dims=(),
                                       collapsed_slice_dims=(0,),
                                       start_index_map=(0,)),
            (1,), mode=lax.GatherScatterMode.PROMISE_IN_BOUNDS)

    rot_m = [(lane + d0) & 31 for d0 in range(D)]
    rot_low = [m & 15 for m in rot_m]
    rot_hi = [m >= 16 for m in rot_m]

    def compute_chunk(c, nrows_v, dust_v, sqst_v):
        def user_body(j, carry):
            urow = c * CH_USERS + j
            u0 = urows_v[urow, pl.ds(0, 16)]
            u1 = urows_v[urow, pl.ds(16, 16)]
            uvec = [jnp.where(rot_hi[d0], _perm(u1, rot_low[d0]),
                              _perm(u0, rot_low[d0])) for d0 in range(D)]
            jb = j * NEG

            def grp_body(g, carry):
                gb = jnp.minimum(g * 16, NEG - 16)
                rb = jb + gb
                rowbase = (rb + lane) * D
                accd = jnp.zeros((16,), _f32)
                accq = jnp.zeros((16,), _f32)
                zero16 = jnp.zeros((16,), _i32)
                for d0 in range(D):
                    col = plsc.load_gather(
                        nrows_v, [zero16, rowbase + rot_m[d0]])
                    accd = accd + uvec[d0] * col
                    accq = accq + col * col
                dust_v[pl.ds(rb, 16)] = accd
                sqst_v[pl.ds(rb, 16)] = accq
                return carry

            return lax.fori_loop(0, 13, grp_body, carry)

        lax.fori_loop(0, CH_USERS, user_body, 0)

    start_gather(0, nrows_a, gsem_a)

    def pair_body(t, carry):
        c0 = 2 * t
        c1 = c0 + 1
        start_gather(c1, nrows_b, gsem_b)
        wait_gather(nrows_a, gsem_a)

        @pl.when(t > 0)
        def _():
            wait_out(dust_a, sqst_a, osem_a)

        compute_chunk(c0, nrows_a, dust_a, sqst_a)
        start_out(c0, dust_a, sqst_a, osem_a)

        @pl.when(t < NCH // 2 - 1)
        def _():
            start_gather(c0 + 2, nrows_a, gsem_a)

        wait_gather(nrows_b, gsem_b)

        @pl.when(t > 0)
        def _():
            wait_out(dust_b, sqst_b, osem_b)

        compute_chunk(c1, nrows_b, dust_b, sqst_b)
        start_out(c1, dust_b, sqst_b, osem_b)
        return carry

    lax.fori_loop(0, NCH // 2, pair_body, 0)
    wait_out(dust_a, sqst_a, osem_a)
    wait_out(dust_b, sqst_b, osem_b)

    def accum_rows(buf, nrows, acc):
        def rb_(k, acc):
            for r8 in range(8):
                a = buf[k * 8 + r8, pl.ds(0, 16)]
                b = buf[k * 8 + r8, pl.ds(16, 16)]
                acc = acc + a * a
                acc = acc + b * b
            return acc

        return lax.fori_loop(0, nrows // 8, rb_, acc)

    def start_sweep(tab_h, row, chunk, buf, gsem):
        pltpu.make_async_copy(tab_h.at[pl.ds(row, chunk)],
                              buf.at[pl.ds(0, chunk)], gsem).start()

    def wait_sweep(tab_h, chunk, buf, gsem):
        pltpu.make_async_copy(tab_h.at[pl.ds(0, chunk)],
                              buf.at[pl.ds(0, chunk)], gsem).wait()

    def sweep_pair(tab_h, row0, chunk, npairs, acc):
        start_sweep(tab_h, row0, chunk, nrows_a, gsem_a)

        def pb(t, acc):
            start_sweep(tab_h, row0 + (2 * t + 1) * chunk, chunk, nrows_b,
                        gsem_b)
            wait_sweep(tab_h, chunk, nrows_a, gsem_a)
            acc = accum_rows(nrows_a, chunk, acc)

            @pl.when(t < npairs - 1)
            def _():
                start_sweep(tab_h, row0 + (2 * t + 2) * chunk, chunk,
                            nrows_a, gsem_a)

            wait_sweep(tab_h, chunk, nrows_b, gsem_b)
            acc = accum_rows(nrows_b, chunk, acc)
            return acc

        return lax.fori_loop(0, npairs, pb, acc)

    acc = jnp.zeros((16,), _f32)
    acc = sweep_pair(itab_h, wid * IT_PW, IT_CH, IT_NP, acc)
    acc = sweep_pair(utab_h, wid * UT_PW, UT_CH, UT_NP, acc)
    pltpu.sync_copy(itab_h.at[pl.ds(IT_TAIL_BASE, IT_TAIL)],
                    nrows_a.at[pl.ds(0, IT_TAIL)])
    t_it = accum_rows(nrows_a, IT_TAIL, jnp.zeros((16,), _f32))
    acc = acc + jnp.where(wid == 0, t_it, jnp.zeros((16,), _f32))
    pltpu.sync_copy(utab_h.at[pl.ds(UT_TAIL_BASE, UT_TAIL)],
                    nrows_a.at[pl.ds(0, UT_TAIL)])
    t_ut = accum_rows(nrows_a, UT_TAIL, jnp.zeros((16,), _f32))
    acc = acc + jnp.where(wid == 1, t_ut, jnp.zeros((16,), _f32))
    part_v[pl.ds(0, 16)] = acc
    pltpu.sync_copy(part_v, ss_o.at[wid])


def _sc_part(users, pos, negf, user_table, item_table, user_degree,
             item_degree):
    mesh = plsc.VectorSubcoreMesh(core_axis_name="c", subcore_axis_name="s",
                                  num_cores=NC, num_subcores=NS)
    kern = pl.kernel(
        _sc_body,
        out_type=(
            jax.ShapeDtypeStruct((B, D), _f32),
            jax.ShapeDtypeStruct((B, D), _f32),
            jax.ShapeDtypeStruct((B * NEG,), _f32),
            jax.ShapeDtypeStruct((B * NEG,), _f32),
            jax.ShapeDtypeStruct((B,), _f32),
            jax.ShapeDtypeStruct((B,), _f32),
            jax.ShapeDtypeStruct((NW, 16), _f32),
        ),
        mesh=mesh,
        compiler_params=pltpu.CompilerParams(needs_layout_passes=False,
                                             use_tc_tiling_on_sc=False),
        scratch_types=[
            pltpu.VMEM((UPW,), _i32),
            pltpu.VMEM((UPW,), _i32),
            pltpu.VMEM((UPW, D), _f32),
            pltpu.VMEM((UPW, D), _f32),
            pltpu.VMEM((UPW,), _f32),
            pltpu.VMEM((UPW,), _f32),
            pltpu.VMEM((UPW * NEG,), _i32),
            pltpu.VMEM((CH_ROWS, D), _f32),
            pltpu.VMEM((CH_ROWS, D), _f32),
            pltpu.VMEM((CH_ROWS,), _f32),
            pltpu.VMEM((CH_ROWS,), _f32),
            pltpu.VMEM((CH_ROWS,), _f32),
            pltpu.VMEM((CH_ROWS,), _f32),
            pltpu.VMEM((16,), _f32),
            pltpu.SemaphoreType.DMA,
            pltpu.SemaphoreType.DMA,
            pltpu.SemaphoreType.DMA,
            pltpu.SemaphoreType.DMA,
            pltpu.SemaphoreType.DMA,
        ],
    )
    return kern(users, pos, negf, user_table, item_table, user_degree,
                item_degree)


RED_GRID = 125


def _min_body(id_ref, ud_ref, mnu_ref, mni_ref):
    i = pl.program_id(0)
    mi = jnp.min(id_ref[...])
    mu = jnp.min(ud_ref[...])

    @pl.when(i == 0)
    def _():
        mni_ref[0, 0] = mi
        mnu_ref[0, 0] = mu

    @pl.when(i > 0)
    def _():
        mni_ref[0, 0] = jnp.minimum(mni_ref[0, 0], mi)
        mnu_ref[0, 0] = jnp.minimum(mnu_ref[0, 0], mu)


def _tc_minred(user_degree, item_degree):
    id2 = item_degree.reshape(RED_GRID, 1, ITEM_NUM // RED_GRID)
    ud2 = user_degree.reshape(RED_GRID, 1, USER_NUM // RED_GRID)
    return pl.pallas_call(
        _min_body,
        grid=(RED_GRID,),
        in_specs=[
            pl.BlockSpec((1, 1, ITEM_NUM // RED_GRID), lambda i: (i, 0, 0)),
            pl.BlockSpec((1, 1, USER_NUM // RED_GRID), lambda i: (i, 0, 0)),
        ],
        out_specs=[
            pl.BlockSpec((1, 1), lambda i: (0, 0), memory_space=pltpu.SMEM),
            pl.BlockSpec((1, 1), lambda i: (0, 0), memory_space=pltpu.SMEM),
        ],
        out_shape=[jax.ShapeDtypeStruct((1, 1), _f32)] * 2,
    )(id2, ud2)


def _loss_body(du_ref, sq_ref, u_ref, p_ref, ud_ref, pd_ref, ss_ref,
               mnu_ref, mni_ref, tot_ref, l1_ref, l2_ref, reg_ref):
    u = u_ref[...]
    p = p_ref[...]
    squ = jnp.sum(u * u, axis=1, keepdims=True)
    dup = jnp.sum(u * p, axis=1, keepdims=True)
    sqp = jnp.sum(p * p, axis=1, keepdims=True)
    cu = jnp.maximum(jnp.sqrt(squ), 1e-12)
    cp = jnp.maximum(jnp.sqrt(sqp), 1e-12)
    du = du_ref[...]
    sq = sq_ref[...]
    cn = jnp.maximum(jnp.sqrt(sq), 1e-12)
    pos_sc = dup / cu
    npos_sc = dup / (cu * cp)
    neg_sc = du / cu
    nneg_sc = du / (cu * cn)
    udeg = ud_ref[...]
    pdeg = pd_ref[...]
    upw = jnp.log(udeg * 1000.0)
    pw = jnp.log(pdeg * 1000.0)
    minu = mnu_ref[0, 0]
    mini = mni_ref[0, 0]
    npw = pw / (-jnp.log(mini * 1000.0 + 1e-7))
    nuw = upw / (-jnp.log(minu * 1000.0 + 1e-7))

    ep1 = jnp.exp((pos_sc + pw + upw) / MARGIN1)
    en1 = jnp.exp(neg_sc / MARGIN1)
    ns1 = jnp.mean(en1, axis=1, keepdims=True)
    d1 = NEG * ns1 + ep1 + 1e-7
    l1 = -jnp.mean(jnp.log(ep1 / d1))

    ep2 = jnp.exp((npos_sc + npw + nuw) / MARGIN2)
    en2 = jnp.exp(nneg_sc / MARGIN2)
    ns2 = jnp.mean(en2, axis=1, keepdims=True)
    d2 = NEG * ns2 + ep2 + 1e-7
    l2 = -jnp.mean(jnp.log(ep2 / d2))

    reg = GAMMA * jnp.sum(ss_ref[...]) / 2.0
    l1w = WEIGHT * l1
    tot_ref[0, 0] = l1w + l2 + reg
    l1_ref[0, 0] = l1w
    l2_ref[0, 0] = l2
    reg_ref[0, 0] = reg


def kernel(users, pos_items, neg_items, user_table, item_table, user_degree,
           item_degree):
    users = users.astype(_i32)
    pos = pos_items.astype(_i32)
    negf = neg_items.astype(_i32).reshape(-1)
    (urows, prows, du, sq, udeg, pdeg, ss) = _sc_part(
        users, pos, negf, user_table, item_table, user_degree, item_degree)
    mnu, mni = _tc_minred(user_degree, item_degree)
    du2 = du.reshape(B, NEG)
    sq2 = sq.reshape(B, NEG)
    out = pl.pallas_call(
        _loss_body,
        in_specs=[
            pl.BlockSpec((B, NEG), lambda: (0, 0)),
            pl.BlockSpec((B, NEG), lambda: (0, 0)),
            pl.BlockSpec((B, D), lambda: (0, 0)),
            pl.BlockSpec((B, D), lambda: (0, 0)),
            pl.BlockSpec((B, 1), lambda: (0, 0)),
            pl.BlockSpec((B, 1), lambda: (0, 0)),
            pl.BlockSpec((NW, 16), lambda: (0, 0)),
            pl.BlockSpec((1, 1), lambda: (0, 0), memory_space=pltpu.SMEM),
            pl.BlockSpec((1, 1), lambda: (0, 0), memory_space=pltpu.SMEM),
        ],
        out_shape=[jax.ShapeDtypeStruct((1, 1), _f32)] * 4,
        out_specs=[pl.BlockSpec((1, 1), lambda: (0, 0),
                                memory_space=pltpu.SMEM)] * 4,
    )(du2, sq2, urows, prows, udeg.reshape(B, 1), pdeg.reshape(B, 1),
      ss, mnu, mni)
    tot, l1w, l2, reg = out
    return (tot[0, 0], l1w[0, 0], l2[0, 0], reg[0, 0])
